# baseline (speedup 1.0000x reference)
import jax
import jax.numpy as jnp
from jax import lax
from jax.experimental import pallas as pl
from jax.experimental.pallas import tpu as pltpu
from jax.experimental.pallas import tpu_sc as plsc

_N = 10000
_E = 320000
_D = 128
_F = 128

_NC = 2
_NS = 16
_NW = _NC * _NS
_EPW = _E // _NW
_WIN = 80
_NWIN = _EPW // _WIN
_OCH = 400
_NOCH = _N // _OCH
_DCH = 2000


def _mesh():
    return plsc.VectorSubcoreMesh(core_axis_name="c", subcore_axis_name="s")


def _deg_body(dst_hbm, deg0, deg1, didx_st, ones_v, zb, acc, sem_a, sem_b):
    cid = lax.axis_index("c")
    sid = lax.axis_index("s")
    wid = sid * _NC + cid

    def fill_ones(i, c):
        ones_v[pl.ds(i * 16, 16)] = jnp.ones((16,), jnp.float32)
        return c

    lax.fori_loop(0, _WIN // 16, fill_ones, 0)

    def fill_z(i, c):
        zb[pl.ds(i * 16, 16)] = jnp.zeros((16,), jnp.float32)
        return c

    lax.fori_loop(0, _DCH // 16, fill_z, 0)

    pltpu.sync_copy(dst_hbm.at[wid], didx_st)

    @pl.when(sid == 0)
    def _():
        def zc(k, c):
            pltpu.sync_copy(zb, acc.at[pl.ds(k * _DCH, _DCH)])
            return c

        lax.fori_loop(0, _N // _DCH, zc, 0)

    plsc.subcore_barrier()

    def fire(w, sem):
        return pltpu.async_copy(ones_v, acc.at[didx_st.at[w]], sem, add=True)

    def drain(sem):
        pltpu.make_async_copy(ones_v, acc.at[didx_st.at[0]], sem).wait()

    fire(0, sem_a)

    def pair(g, c):
        w0 = 2 * g
        fire(w0 + 1, sem_b)
        drain(sem_a)
        fire(w0 + 2, sem_a)
        drain(sem_b)
        return c

    lax.fori_loop(0, (_NWIN - 1) // 2, pair, 0)
    drain(sem_a)

    plsc.subcore_barrier()

    @pl.when(sid == 0)
    def _():
        @pl.when(cid == 0)
        def _():
            pltpu.sync_copy(acc, deg0)

        @pl.when(cid == 1)
        def _():
            pltpu.sync_copy(acc, deg1)


def _degrees(dst):
    k = pl.kernel(
        _deg_body,
        out_type=[
            jax.ShapeDtypeStruct((_N,), jnp.float32),
            jax.ShapeDtypeStruct((_N,), jnp.float32),
        ],
        mesh=_mesh(),
        scratch_types=[
            pltpu.VMEM((_NWIN, _WIN), jnp.int32),
            pltpu.VMEM((_WIN,), jnp.float32),
            pltpu.VMEM((_DCH,), jnp.float32),
            pltpu.VMEM_SHARED((_N,), jnp.float32),
            pltpu.SemaphoreType.DMA,
            pltpu.SemaphoreType.DMA,
        ],
    )
    return k(dst)


def _agg_body(src_hbm, dst_hbm, h1_hbm, p0, p1,
              si0, si1, si2, si3, di0, di1, di2, di3,
              r0, r1, r2, r3, acc,
              mi0, mi1, mi2, mi3, mg0, mg1, mg2, mg3,
              ms0, ms1, ms2, ms3):
    cid = lax.axis_index("c")
    sid = lax.axis_index("s")
    wid = sid * _NC + cid

    sidx = (si0, si1, si2, si3)
    didx = (di0, di1, di2, di3)
    rows = (r0, r1, r2, r3)
    semi = (mi0, mi1, mi2, mi3)
    semg = (mg0, mg1, mg2, mg3)
    sems = (ms0, ms1, ms2, ms3)

    def fill_z(i, c):
        r = i // (_D // 16)
        q = i % (_D // 16)
        r0[r, pl.ds(q * 16, 16)] = jnp.zeros((16,), jnp.float32)
        return c

    lax.fori_loop(0, _WIN * (_D // 16), fill_z, 0)

    for rnd in range(2):
        ch = sid + rnd * _NS

        @pl.when(ch < _NOCH)
        def _():
            def zc(j, c):
                pltpu.sync_copy(r0, acc.at[pl.ds(ch * _OCH + j * _WIN, _WIN)])
                return c

            lax.fori_loop(0, _OCH // _WIN, zc, 0)

    plsc.subcore_barrier()

    base = wid * _EPW

    def fire_i(j, b):
        off = base + j * _WIN
        pltpu.async_copy(src_hbm.at[pl.ds(off, _WIN)], sidx[b], semi[b])
        pltpu.async_copy(dst_hbm.at[pl.ds(off, _WIN)], didx[b], semi[b])

    def wait_i(b):
        pltpu.make_async_copy(src_hbm.at[pl.ds(0, _WIN)], sidx[b],
                              semi[b]).wait()
        pltpu.make_async_copy(dst_hbm.at[pl.ds(0, _WIN)], didx[b],
                              semi[b]).wait()

    def fire_g(j, b):
        pltpu.async_copy(h1_hbm.at[sidx[b]], rows[b], semg[b])

    def wait_g(b):
        pltpu.make_async_copy(h1_hbm.at[sidx[b]], rows[b], semg[b]).wait()

    def fire_s(j, b):
        pltpu.async_copy(rows[b], acc.at[didx[b]], sems[b], add=True)

    def wait_s(b):
        pltpu.make_async_copy(rows[b], acc.at[didx[b]], sems[b]).wait()

    fire_i(0, 0)
    wait_i(0)
    fire_g(0, 0)
    fire_i(1, 1)

    wait_g(0)
    fire_s(0, 0)
    wait_i(1)
    fire_g(1, 1)
    fire_i(2, 2)

    wait_g(1)
    fire_s(1, 1)
    wait_i(2)
    fire_g(2, 2)
    fire_i(3, 3)

    def steady(g, c):
        j = 4 * g + 2
        for t, (b, b1, b2) in enumerate(
                ((2, 3, 0), (3, 0, 1), (0, 1, 2), (1, 2, 3))):
            wait_g(b)
            fire_s(j + t, b)
            wait_i(b1)
            fire_g(j + t + 1, b1)
            wait_s(b2)
            fire_i(j + t + 2, b2)
        return c

    lax.fori_loop(0, (_NWIN - 5) // 4, steady, 0)

    wait_g(2)
    fire_s(_NWIN - 3, 2)
    wait_i(3)
    fire_g(_NWIN - 2, 3)
    wait_s(0)
    fire_i(_NWIN - 1, 0)

    wait_g(3)
    fire_s(_NWIN - 2, 3)
    wait_i(0)
    fire_g(_NWIN - 1, 0)
    wait_s(1)

    wait_g(0)
    fire_s(_NWIN - 1, 0)

    wait_s(2)
    wait_s(3)
    wait_s(0)

    plsc.subcore_barrier()

    for rnd in range(2):
        ch = sid + rnd * _NS

        @pl.when(ch < _NOCH)
        def _():
            sl = pl.ds(ch * _OCH, _OCH)

            @pl.when(cid == 0)
            def _():
                pltpu.sync_copy(acc.at[sl], p0.at[sl])

            @pl.when(cid == 1)
            def _():
                pltpu.sync_copy(acc.at[sl], p1.at[sl])


def _aggregate(src, dst, h1):
    k = pl.kernel(
        _agg_body,
        out_type=[
            jax.ShapeDtypeStruct((_N, _F), jnp.float32),
            jax.ShapeDtypeStruct((_N, _F), jnp.float32),
        ],
        mesh=_mesh(),
        scratch_types=[
            pltpu.VMEM((_WIN,), jnp.int32),
            pltpu.VMEM((_WIN,), jnp.int32),
            pltpu.VMEM((_WIN,), jnp.int32),
            pltpu.VMEM((_WIN,), jnp.int32),
            pltpu.VMEM((_WIN,), jnp.int32),
            pltpu.VMEM((_WIN,), jnp.int32),
            pltpu.VMEM((_WIN,), jnp.int32),
            pltpu.VMEM((_WIN,), jnp.int32),
            pltpu.VMEM((_WIN, _F), jnp.float32),
            pltpu.VMEM((_WIN, _F), jnp.float32),
            pltpu.VMEM((_WIN, _F), jnp.float32),
            pltpu.VMEM((_WIN, _F), jnp.float32),
            pltpu.VMEM_SHARED((_N, _F), jnp.float32),
            pltpu.SemaphoreType.DMA,
            pltpu.SemaphoreType.DMA,
            pltpu.SemaphoreType.DMA,
            pltpu.SemaphoreType.DMA,
            pltpu.SemaphoreType.DMA,
            pltpu.SemaphoreType.DMA,
            pltpu.SemaphoreType.DMA,
            pltpu.SemaphoreType.DMA,
            pltpu.SemaphoreType.DMA,
            pltpu.SemaphoreType.DMA,
            pltpu.SemaphoreType.DMA,
            pltpu.SemaphoreType.DMA,
        ],
    )
    return k(src, dst, h1)


def _proj_body(h_ref, w_ref, d0_ref, d1_ref, o_ref):
    deg = d0_ref[...] + d1_ref[...]
    nrm = lax.rsqrt(deg)
    hs = h_ref[...] * nrm[:, None]
    o_ref[...] = jnp.dot(
        hs, w_ref[...],
        preferred_element_type=jnp.float32,
        precision=lax.Precision.HIGHEST,
    )


def _project(h, W, d0, d1):
    return pl.pallas_call(
        _proj_body,
        out_shape=jax.ShapeDtypeStruct((_N, _F), jnp.float32),
    )(h, W, d0, d1)


def _fin_body(p0_ref, p1_ref, d0_ref, d1_ref, o_ref):
    deg = d0_ref[...] + d1_ref[...]
    nrm = lax.rsqrt(deg)
    o_ref[...] = (p0_ref[...] + p1_ref[...]) * nrm[:, None]


def _finalize(p0, p1, d0, d1):
    return pl.pallas_call(
        _fin_body,
        out_shape=jax.ShapeDtypeStruct((_N, _F), jnp.float32),
    )(p0, p1, d0, d1)


def kernel(edge_index, h, W):
    dst = edge_index[0].astype(jnp.int32)
    src = edge_index[1].astype(jnp.int32)
    d0, d1 = _degrees(dst.reshape(_NW, _NWIN, _WIN))
    h1 = _project(h, W, d0, d1)
    p0, p1 = _aggregate(src, dst, h1)
    return _finalize(p0, p1, d0, d1)


# speedup vs baseline: 1.2698x; 1.2698x over previous
import functools

import jax
import jax.numpy as jnp
from jax import lax
from jax.experimental import pallas as pl
from jax.experimental.pallas import tpu as pltpu
from jax.experimental.pallas import tpu_sc as plsc

_N = 10000
_E = 320000
_D = 128
_F = 128

_NC = 2
_NS = 16
_NW = _NC * _NS
_EPW = _E // _NW
_WIN = 80
_NWIN = _EPW // _WIN
_NMAC = 5
_MWIN = _NWIN // _NMAC
_OCH = 400
_NOCH = _N // _OCH
_ZCH = 80
_DCH = 2000


def _mesh():
    return plsc.VectorSubcoreMesh(core_axis_name="c", subcore_axis_name="s")


def _deg_body(dst_hbm, deg0, deg1, didx_st, ones_v, zb, acc, sem_a, sem_b):
    cid = lax.axis_index("c")
    sid = lax.axis_index("s")
    wid = sid * _NC + cid

    def fill_ones(i, c):
        ones_v[pl.ds(i * 16, 16)] = jnp.ones((16,), jnp.float32)
        return c

    lax.fori_loop(0, _WIN // 16, fill_ones, 0)

    def fill_z(i, c):
        zb[pl.ds(i * 16, 16)] = jnp.zeros((16,), jnp.float32)
        return c

    lax.fori_loop(0, _DCH // 16, fill_z, 0)

    pltpu.sync_copy(dst_hbm.at[wid], didx_st)

    @pl.when(sid == 0)
    def _():
        def zc(k, c):
            pltpu.sync_copy(zb, acc.at[pl.ds(k * _DCH, _DCH)])
            return c

        lax.fori_loop(0, _N // _DCH, zc, 0)

    plsc.subcore_barrier()

    def fire(w, sem):
        return pltpu.async_copy(
            ones_v, acc.at[didx_st.at[w // _MWIN, w % _MWIN]], sem, add=True)

    def drain(sem):
        pltpu.make_async_copy(ones_v, acc.at[didx_st.at[0, 0]], sem).wait()

    fire(0, sem_a)

    def pair(g, c):
        w0 = 2 * g
        fire(w0 + 1, sem_b)
        drain(sem_a)
        fire(w0 + 2, sem_a)
        drain(sem_b)
        return c

    lax.fori_loop(0, (_NWIN - 1) // 2, pair, 0)
    drain(sem_a)

    plsc.subcore_barrier()

    @pl.when(sid == 0)
    def _():
        @pl.when(cid == 0)
        def _():
            pltpu.sync_copy(acc, deg0)

        @pl.when(cid == 1)
        def _():
            pltpu.sync_copy(acc, deg1)


def _degrees(dst):
    k = pl.kernel(
        _deg_body,
        out_type=[
            jax.ShapeDtypeStruct((_N,), jnp.float32),
            jax.ShapeDtypeStruct((_N,), jnp.float32),
        ],
        mesh=_mesh(),
        scratch_types=[
            pltpu.VMEM((_NMAC, _MWIN, _WIN), jnp.int32),
            pltpu.VMEM((_WIN,), jnp.float32),
            pltpu.VMEM((_DCH,), jnp.float32),
            pltpu.VMEM_SHARED((_N,), jnp.float32),
            pltpu.SemaphoreType.DMA,
            pltpu.SemaphoreType.DMA,
        ],
    )
    return k(dst)


def _agg_body(src_hbm, dst_hbm, h1_hbm, p0, p1, sidx_st, didx_st, rows_a,
              rows_b, rows_c, acc, sg_a, sg_b, sg_c, ss_a, ss_b, ss_c):
    cid = lax.axis_index("c")
    sid = lax.axis_index("s")
    wid = sid * _NC + cid

    def fill_z(i, c):
        r = i // (_D // 16)
        q = i % (_D // 16)
        rows_a[r, pl.ds(q * 16, 16)] = jnp.zeros((16,), jnp.float32)
        return c

    lax.fori_loop(0, _ZCH * (_D // 16), fill_z, 0)

    for rnd in range(2):
        ch = sid + rnd * _NS

        @pl.when(ch < _NOCH)
        def _():
            def zc(j, c):
                pltpu.sync_copy(
                    rows_a, acc.at[pl.ds(ch * _OCH + j * _ZCH, _ZCH)])
                return c

            lax.fori_loop(0, _OCH // _ZCH, zc, 0)

    plsc.subcore_barrier()

    bufs = (rows_a, rows_b, rows_c)
    gsem = (sg_a, sg_b, sg_c)
    ssem = (ss_a, ss_b, ss_c)

    def fire_g(j, b):
        pltpu.async_copy(h1_hbm.at[sidx_st.at[j]], bufs[b], gsem[b])

    def wait_g(b):
        pltpu.make_async_copy(h1_hbm.at[sidx_st.at[0]], bufs[b],
                              gsem[b]).wait()

    def fire_s(j, b):
        pltpu.async_copy(bufs[b], acc.at[didx_st.at[j]], ssem[b], add=True)

    def wait_s(b):
        pltpu.make_async_copy(bufs[b], acc.at[didx_st.at[0]], ssem[b]).wait()

    for m in range(_NMAC):
        pltpu.sync_copy(src_hbm.at[wid, m], sidx_st)
        pltpu.sync_copy(dst_hbm.at[wid, m], didx_st)

        fire_g(0, 0)
        fire_g(1, 1)
        wait_g(0)
        fire_s(0, 0)
        fire_g(2, 2)
        wait_g(1)
        fire_s(1, 1)
        wait_s(0)
        fire_g(3, 0)

        def steady(g, c):
            j = 3 * g + 2
            for t, (b, b2) in enumerate(((2, 1), (0, 2), (1, 0))):
                wait_g(b)
                fire_s(j + t, b)
                wait_s(b2)
                fire_g(j + t + 2, b2)
            return c

        lax.fori_loop(0, (_MWIN - 4) // 3, steady, 0)

        wait_g(2)
        fire_s(_MWIN - 2, 2)
        wait_g(0)
        fire_s(_MWIN - 1, 0)
        wait_s(1)
        wait_s(2)
        wait_s(0)

    plsc.subcore_barrier()

    for rnd in range(2):
        ch = sid + rnd * _NS

        @pl.when(ch < _NOCH)
        def _():
            sl = pl.ds(ch * _OCH, _OCH)

            @pl.when(cid == 0)
            def _():
                pltpu.sync_copy(acc.at[sl], p0.at[sl])

            @pl.when(cid == 1)
            def _():
                pltpu.sync_copy(acc.at[sl], p1.at[sl])


def _aggregate(src, dst, h1):
    k = pl.kernel(
        _agg_body,
        out_type=[
            jax.ShapeDtypeStruct((_N, _F), jnp.float32),
            jax.ShapeDtypeStruct((_N, _F), jnp.float32),
        ],
        mesh=_mesh(),
        scratch_types=[
            pltpu.VMEM((_MWIN, _WIN), jnp.int32),
            pltpu.VMEM((_MWIN, _WIN), jnp.int32),
            pltpu.VMEM((_WIN, _F), jnp.float32),
            pltpu.VMEM((_WIN, _F), jnp.float32),
            pltpu.VMEM((_WIN, _F), jnp.float32),
            pltpu.VMEM_SHARED((_N, _F), jnp.float32),
            pltpu.SemaphoreType.DMA,
            pltpu.SemaphoreType.DMA,
            pltpu.SemaphoreType.DMA,
            pltpu.SemaphoreType.DMA,
            pltpu.SemaphoreType.DMA,
            pltpu.SemaphoreType.DMA,
        ],
    )
    return k(src, dst, h1)


_B = 512


def _proj_body(h_ref, w_ref, d0_ref, d1_ref, o_ref):
    deg = d0_ref[...] + d1_ref[...]
    nrm = lax.rsqrt(deg)
    hs = h_ref[...] * nrm[:, None]
    o_ref[...] = jnp.dot(
        hs, w_ref[...],
        preferred_element_type=jnp.float32,
        precision=lax.Precision.HIGHEST,
    )


def _project(h, W, d0, d1):
    return pl.pallas_call(
        _proj_body,
        out_shape=jax.ShapeDtypeStruct((_N, _F), jnp.float32),
    )(h, W, d0, d1)


def _fin_body(p0_ref, p1_ref, d0_ref, d1_ref, o_ref):
    deg = d0_ref[...] + d1_ref[...]
    nrm = lax.rsqrt(deg)
    o_ref[...] = (p0_ref[...] + p1_ref[...]) * nrm[:, None]


def _finalize(p0, p1, d0, d1):
    return pl.pallas_call(
        _fin_body,
        out_shape=jax.ShapeDtypeStruct((_N, _F), jnp.float32),
    )(p0, p1, d0, d1)


def kernel(edge_index, h, W):
    dst = edge_index[0].astype(jnp.int32).reshape(_NW, _NMAC, _MWIN, _WIN)
    src = edge_index[1].astype(jnp.int32).reshape(_NW, _NMAC, _MWIN, _WIN)
    d0, d1 = _degrees(dst)
    h1 = _project(h, W, d0, d1)
    p0, p1 = _aggregate(src, dst, h1)
    return _finalize(p0, p1, d0, d1)
